# Initial kernel scaffold; baseline (speedup 1.0000x reference)
import jax
import jax.numpy as jnp
from jax.experimental import pallas as pl


def kernel(nfeats, efeats, edge_index, W_apply_w, W_apply_b, W_edge_w, W_edge_b):
    raise NotImplementedError("write your pallas kernel here")


# speedup vs baseline: 2.6067x; 2.6067x over previous
import jax
import jax.numpy as jnp
from jax import lax
from jax.experimental import pallas as pl
from jax.experimental.pallas import tpu as pltpu
from jax.experimental.pallas import tpu_sc as plsc

N = 10000
E = 320000
D_IN = 128
D_E = 16
D_OUT = 128
D_EDGE = 256

NC = 2
NS = 16
NW = NC * NS
EPW = E // NW
CHUNK = 80
NCHUNK = EPW // CHUNK
RPT = 640
N_PAD = NS * RPT
ACC_W = 32


ZCH = RPT // CHUNK


def _seg_body(efpad_hbm, dst_hbm, parts_hbm, idx_v, pad_v, zidx_v, acc_sh):
    c = lax.axis_index("c")
    s = lax.axis_index("s")
    wid = c * NS + s

    iota16 = lax.iota(jnp.int32, 16)
    base = s * RPT
    for k in range(ZCH):
        for b in range(CHUNK // 16):
            zidx_v[k, pl.ds(b * 16, 16)] = base + (k * CHUNK + b * 16) + iota16

    zeros16 = jnp.zeros((16,), jnp.float32)

    def zrow(r, carry):
        pad_v[r, pl.ds(0, 16)] = zeros16
        pad_v[r, pl.ds(16, 16)] = zeros16
        return carry

    lax.fori_loop(0, CHUNK, zrow, None)

    def zchunk(k, carry):
        pltpu.sync_copy(pad_v, acc_sh.at[zidx_v.at[k]])
        return carry

    lax.fori_loop(0, ZCH, zchunk, None)
    plsc.subcore_barrier()

    pltpu.sync_copy(dst_hbm.at[wid], idx_v)

    def chunk(j, carry):
        pltpu.sync_copy(efpad_hbm.at[wid, j], pad_v)
        pltpu.sync_copy(pad_v, acc_sh.at[idx_v.at[j]], add=True)
        return carry

    lax.fori_loop(0, NCHUNK, chunk, None)
    plsc.subcore_barrier()

    def wchunk(k, carry):
        pltpu.sync_copy(acc_sh.at[zidx_v.at[k]], pad_v)
        pltpu.sync_copy(pad_v, parts_hbm.at[c, s, k])
        return carry

    lax.fori_loop(0, ZCH, wchunk, None)


def _tc_body(nf_ref, parts_ref, waT_ref, ba_ref, weT_ref, be_ref,
             h_ref, hs_ref, hd_ref):
    p = parts_ref[0] + parts_ref[1]
    ssum = p[:, :D_E]
    cnt = jnp.maximum(p[:, D_E:D_E + 1], 1.0)
    hn = ssum / cnt

    nf = nf_ref[...]
    waT = waT_ref[...]
    hp = lax.Precision.HIGHEST
    z = (jnp.dot(nf, waT[:D_IN, :], precision=hp)
         + jnp.dot(hn, waT[D_IN:, :], precision=hp)
         + ba_ref[...])
    h = jnp.maximum(z, 0.0)
    h_ref[...] = h

    weT = weT_ref[...]
    hs_ref[...] = jnp.dot(h, weT[:D_OUT, :], precision=hp)
    hd_ref[...] = jnp.dot(h, weT[D_OUT:, :], precision=hp) + be_ref[...]


def _edge_body(hs_hbm, hd_hbm, src_hbm, dst_hbm, out_hbm,
               sidx_v, didx_v, bufs_v, bufd_v, sem_s, sem_d):
    c = lax.axis_index("c")
    s = lax.axis_index("s")
    wid = c * NS + s

    pltpu.sync_copy(src_hbm.at[wid], sidx_v)
    pltpu.sync_copy(dst_hbm.at[wid], didx_v)

    def chunk(j, carry):
        cps = pltpu.async_copy(hs_hbm.at[sidx_v.at[j]], bufs_v, sem_s)
        cpd = pltpu.async_copy(hd_hbm.at[didx_v.at[j]], bufd_v, sem_d)
        cps.wait()
        cpd.wait()

        def addrow(r, inner):
            for k in range(D_EDGE // 16):
                sl = pl.ds(k * 16, 16)
                bufs_v[r, sl] = bufs_v[r, sl] + bufd_v[r, sl]
            return inner

        lax.fori_loop(0, CHUNK, addrow, None)
        pltpu.sync_copy(bufs_v, out_hbm.at[wid, j])
        return carry

    lax.fori_loop(0, NCHUNK, chunk, None)


def kernel(nfeats, efeats, edge_index, W_apply_w, W_apply_b, W_edge_w, W_edge_b):
    nf = nfeats.reshape(N, D_IN)
    ef_pad = jnp.concatenate(
        [efeats.reshape(E, D_E),
         jnp.ones((E, 1), jnp.float32),
         jnp.zeros((E, ACC_W - D_E - 1), jnp.float32)],
        axis=1).reshape(NW, NCHUNK, CHUNK, ACC_W)
    src3 = edge_index[0].reshape(NW, NCHUNK, CHUNK)
    dst3 = edge_index[1].reshape(NW, NCHUNK, CHUNK)
    waT = W_apply_w.T
    weT = W_edge_w.T
    ba = W_apply_b.reshape(1, D_OUT)
    be = W_edge_b.reshape(1, D_EDGE)

    mesh = plsc.VectorSubcoreMesh(core_axis_name="c", subcore_axis_name="s")

    seg = pl.kernel(
        _seg_body,
        out_type=jax.ShapeDtypeStruct((NC, NS, ZCH, CHUNK, ACC_W), jnp.float32),
        mesh=mesh,
        scratch_types=[
            pltpu.VMEM((NCHUNK, CHUNK), jnp.int32),
            pltpu.VMEM((CHUNK, ACC_W), jnp.float32),
            pltpu.VMEM((ZCH, CHUNK), jnp.int32),
            pltpu.VMEM_SHARED((N_PAD, ACC_W), jnp.float32),
        ],
    )
    parts = seg(ef_pad, dst3).reshape(NC, N_PAD, ACC_W)[:, :N]

    BLK = 2000
    grid = (N // BLK,)
    h, hs, hd = pl.pallas_call(
        _tc_body,
        grid=grid,
        in_specs=[
            pl.BlockSpec((BLK, D_IN), lambda i: (i, 0)),
            pl.BlockSpec((NC, BLK, ACC_W), lambda i: (0, i, 0)),
            pl.BlockSpec((D_IN + D_E, D_OUT), lambda i: (0, 0)),
            pl.BlockSpec((1, D_OUT), lambda i: (0, 0)),
            pl.BlockSpec((D_EDGE, D_EDGE), lambda i: (0, 0)),
            pl.BlockSpec((1, D_EDGE), lambda i: (0, 0)),
        ],
        out_specs=[
            pl.BlockSpec((BLK, D_OUT), lambda i: (i, 0)),
            pl.BlockSpec((BLK, D_EDGE), lambda i: (i, 0)),
            pl.BlockSpec((BLK, D_EDGE), lambda i: (i, 0)),
        ],
        out_shape=[
            jax.ShapeDtypeStruct((N, D_OUT), jnp.float32),
            jax.ShapeDtypeStruct((N, D_EDGE), jnp.float32),
            jax.ShapeDtypeStruct((N, D_EDGE), jnp.float32),
        ],
    )(nf, parts, waT, ba, weT, be)

    edge = pl.kernel(
        _edge_body,
        out_type=jax.ShapeDtypeStruct((NW, NCHUNK, CHUNK, D_EDGE), jnp.float32),
        mesh=mesh,
        scratch_types=[
            pltpu.VMEM((NCHUNK, CHUNK), jnp.int32),
            pltpu.VMEM((NCHUNK, CHUNK), jnp.int32),
            pltpu.VMEM((CHUNK, D_EDGE), jnp.float32),
            pltpu.VMEM((CHUNK, D_EDGE), jnp.float32),
            pltpu.SemaphoreType.DMA,
            pltpu.SemaphoreType.DMA,
        ],
    )(hs, hd, src3, dst3)

    return h.reshape(N, 1, D_OUT), edge.reshape(E, 1, D_EDGE)


# speedup vs baseline: 2.9103x; 1.1165x over previous
import jax
import jax.numpy as jnp
from jax import lax
from jax.experimental import pallas as pl
from jax.experimental.pallas import tpu as pltpu
from jax.experimental.pallas import tpu_sc as plsc

N = 10000
E = 320000
D_IN = 128
D_E = 16
D_OUT = 128
D_EDGE = 256

NC = 2
NS = 16
NW = NC * NS
EPW = E // NW
CHUNK = 80
NCHUNK = EPW // CHUNK
RPT = 640
N_PAD = NS * RPT
ACC_W = 32


ZCH = RPT // CHUNK


def _seg_body(efpad_hbm, dst_hbm, parts_hbm, idx_v, pad_v, zidx_v, acc_sh):
    c = lax.axis_index("c")
    s = lax.axis_index("s")
    wid = c * NS + s

    iota16 = lax.iota(jnp.int32, 16)
    base = s * RPT
    for k in range(ZCH):
        for b in range(CHUNK // 16):
            zidx_v[k, pl.ds(b * 16, 16)] = base + (k * CHUNK + b * 16) + iota16

    zeros16 = jnp.zeros((16,), jnp.float32)

    def zrow(r, carry):
        pad_v[r, pl.ds(0, 16)] = zeros16
        pad_v[r, pl.ds(16, 16)] = zeros16
        return carry

    lax.fori_loop(0, CHUNK, zrow, None)

    def zchunk(k, carry):
        pltpu.sync_copy(pad_v, acc_sh.at[zidx_v.at[k]])
        return carry

    lax.fori_loop(0, ZCH, zchunk, None)
    plsc.subcore_barrier()

    pltpu.sync_copy(dst_hbm.at[wid], idx_v)

    def chunk(j, carry):
        pltpu.sync_copy(efpad_hbm.at[wid, j], pad_v)
        pltpu.sync_copy(pad_v, acc_sh.at[idx_v.at[j]], add=True)
        return carry

    lax.fori_loop(0, NCHUNK, chunk, None)
    plsc.subcore_barrier()

    def wchunk(k, carry):
        pltpu.sync_copy(acc_sh.at[zidx_v.at[k]], pad_v)
        pltpu.sync_copy(pad_v, parts_hbm.at[c, s, k])
        return carry

    lax.fori_loop(0, ZCH, wchunk, None)


def _tc_body(nf_ref, parts_ref, waT_ref, ba_ref, weT_ref, be_ref,
             h_ref, hs_ref, hd_ref):
    p = parts_ref[0] + parts_ref[1]
    ssum = p[:, :D_E]
    cnt = jnp.maximum(p[:, D_E:D_E + 1], 1.0)
    hn = ssum / cnt

    nf = nf_ref[...]
    waT = waT_ref[...]
    hp = lax.Precision.HIGHEST
    z = (jnp.dot(nf, waT[:D_IN, :], precision=hp)
         + jnp.dot(hn, waT[D_IN:, :], precision=hp)
         + ba_ref[...])
    h = jnp.maximum(z, 0.0)
    h_ref[...] = h

    weT = weT_ref[...]
    hs_ref[...] = jnp.dot(h, weT[:D_OUT, :], precision=hp)
    hd_ref[...] = jnp.dot(h, weT[D_OUT:, :], precision=hp) + be_ref[...]


ECH = 40
NECH = EPW // ECH
EROWS = 2 * ECH


def _edge_body(hs_hbm, hd_hbm, src_hbm, dst_hbm, out_hbm,
               sidx_v, didx_v, bufs_v, bufd_v, sem_s, sem_d):
    c = lax.axis_index("c")
    s = lax.axis_index("s")
    wid = c * NS + s

    pltpu.sync_copy(src_hbm.at[wid], sidx_v)
    pltpu.sync_copy(dst_hbm.at[wid], didx_v)

    def chunk(j, carry):
        cps = pltpu.async_copy(hs_hbm.at[sidx_v.at[j]], bufs_v, sem_s)
        cpd = pltpu.async_copy(hd_hbm.at[didx_v.at[j]], bufd_v, sem_d)
        cps.wait()
        cpd.wait()

        def addrow(r, inner):
            for k in range(128 // 16):
                sl = pl.ds(k * 16, 16)
                bufs_v[r, sl] = bufs_v[r, sl] + bufd_v[r, sl]
            return inner

        lax.fori_loop(0, EROWS, addrow, None)
        pltpu.sync_copy(bufs_v,
                        out_hbm.at[pl.ds(2 * wid * EPW + j * EROWS, EROWS)])
        return carry

    lax.fori_loop(0, NECH, chunk, None)


def kernel(nfeats, efeats, edge_index, W_apply_w, W_apply_b, W_edge_w, W_edge_b):
    nf = nfeats.reshape(N, D_IN)
    ef_pad = jnp.concatenate(
        [efeats.reshape(E, D_E),
         jnp.ones((E, 1), jnp.float32),
         jnp.zeros((E, ACC_W - D_E - 1), jnp.float32)],
        axis=1).reshape(NW, NCHUNK, CHUNK, ACC_W)
    src3 = edge_index[0].reshape(NW, NCHUNK, CHUNK)
    dst3 = edge_index[1].reshape(NW, NCHUNK, CHUNK)
    waT = W_apply_w.T
    weT = W_edge_w.T
    ba = W_apply_b.reshape(1, D_OUT)
    be = W_edge_b.reshape(1, D_EDGE)

    mesh = plsc.VectorSubcoreMesh(core_axis_name="c", subcore_axis_name="s")

    seg = pl.kernel(
        _seg_body,
        out_type=jax.ShapeDtypeStruct((NC, NS, ZCH, CHUNK, ACC_W), jnp.float32),
        mesh=mesh,
        scratch_types=[
            pltpu.VMEM((NCHUNK, CHUNK), jnp.int32),
            pltpu.VMEM((CHUNK, ACC_W), jnp.float32),
            pltpu.VMEM((ZCH, CHUNK), jnp.int32),
            pltpu.VMEM_SHARED((N_PAD, ACC_W), jnp.float32),
        ],
    )
    parts = seg(ef_pad, dst3).reshape(NC, N_PAD, ACC_W)[:, :N]

    BLK = 2000
    grid = (N // BLK,)
    h, hs, hd = pl.pallas_call(
        _tc_body,
        grid=grid,
        in_specs=[
            pl.BlockSpec((BLK, D_IN), lambda i: (i, 0)),
            pl.BlockSpec((NC, BLK, ACC_W), lambda i: (0, i, 0)),
            pl.BlockSpec((D_IN + D_E, D_OUT), lambda i: (0, 0)),
            pl.BlockSpec((1, D_OUT), lambda i: (0, 0)),
            pl.BlockSpec((D_EDGE, D_EDGE), lambda i: (0, 0)),
            pl.BlockSpec((1, D_EDGE), lambda i: (0, 0)),
        ],
        out_specs=[
            pl.BlockSpec((BLK, D_OUT), lambda i: (i, 0)),
            pl.BlockSpec((BLK, D_EDGE), lambda i: (i, 0)),
            pl.BlockSpec((BLK, D_EDGE), lambda i: (i, 0)),
        ],
        out_shape=[
            jax.ShapeDtypeStruct((N, D_OUT), jnp.float32),
            jax.ShapeDtypeStruct((N, D_EDGE), jnp.float32),
            jax.ShapeDtypeStruct((N, D_EDGE), jnp.float32),
        ],
    )(nf, parts, waT, ba, weT, be)

    hs2 = hs.reshape(2 * N, 128)
    hd2 = hd.reshape(2 * N, 128)
    two = jnp.arange(2, dtype=jnp.int32)
    srcx = (2 * edge_index[0][:, None] + two).reshape(NW, NECH, EROWS)
    dstx = (2 * edge_index[1][:, None] + two).reshape(NW, NECH, EROWS)

    edge = pl.kernel(
        _edge_body,
        out_type=jax.ShapeDtypeStruct((2 * E, 128), jnp.float32),
        mesh=mesh,
        scratch_types=[
            pltpu.VMEM((NECH, EROWS), jnp.int32),
            pltpu.VMEM((NECH, EROWS), jnp.int32),
            pltpu.VMEM((EROWS, 128), jnp.float32),
            pltpu.VMEM((EROWS, 128), jnp.float32),
            pltpu.SemaphoreType.DMA,
            pltpu.SemaphoreType.DMA,
        ],
    )(hs2, hd2, srcx, dstx)

    return h.reshape(N, 1, D_OUT), edge.reshape(E, 1, D_EDGE)


# speedup vs baseline: 3.6393x; 1.2505x over previous
import jax
import jax.numpy as jnp
from jax import lax
from jax.experimental import pallas as pl
from jax.experimental.pallas import tpu as pltpu
from jax.experimental.pallas import tpu_sc as plsc

N = 10000
E = 320000
D_IN = 128
D_E = 16
D_OUT = 128
D_EDGE = 256

NC = 2
NS = 16
NW = NC * NS
EPW = E // NW
CHUNK = 80
NCHUNK = EPW // CHUNK
RPT = 640
N_PAD = NS * RPT
ACC_W = 32


ZCH = RPT // CHUNK


def _seg_body(efpad_hbm, dst_hbm, parts_hbm, idx_v, pad_v, zidx_v, acc_sh):
    c = lax.axis_index("c")
    s = lax.axis_index("s")
    wid = c * NS + s

    iota16 = lax.iota(jnp.int32, 16)
    base = s * RPT
    for k in range(ZCH):
        for b in range(CHUNK // 16):
            zidx_v[k, pl.ds(b * 16, 16)] = base + (k * CHUNK + b * 16) + iota16

    zeros16 = jnp.zeros((16,), jnp.float32)

    def zrow(r, carry):
        pad_v[r, pl.ds(0, 16)] = zeros16
        pad_v[r, pl.ds(16, 16)] = zeros16
        return carry

    lax.fori_loop(0, CHUNK, zrow, None)

    def zchunk(k, carry):
        pltpu.sync_copy(pad_v, acc_sh.at[zidx_v.at[k]])
        return carry

    lax.fori_loop(0, ZCH, zchunk, None)
    plsc.subcore_barrier()

    pltpu.sync_copy(dst_hbm.at[wid], idx_v)

    def chunk(j, carry):
        pltpu.sync_copy(efpad_hbm.at[wid, j], pad_v)
        pltpu.sync_copy(pad_v, acc_sh.at[idx_v.at[j]], add=True)
        return carry

    lax.fori_loop(0, NCHUNK, chunk, None)
    plsc.subcore_barrier()

    def wchunk(k, carry):
        pltpu.sync_copy(acc_sh.at[zidx_v.at[k]], pad_v)
        pltpu.sync_copy(pad_v, parts_hbm.at[c, s, k])
        return carry

    lax.fori_loop(0, ZCH, wchunk, None)


def _tc_body(nf_ref, parts_ref, waT_ref, ba_ref, weT_ref, be_ref,
             h_ref, hs_ref, hd_ref):
    p = parts_ref[0] + parts_ref[1]
    ssum = p[:, :D_E]
    cnt = jnp.maximum(p[:, D_E:D_E + 1], 1.0)
    hn = ssum / cnt

    nf = nf_ref[...]
    waT = waT_ref[...]
    hp = lax.Precision.HIGHEST
    z = (jnp.dot(nf, waT[:D_IN, :], precision=hp)
         + jnp.dot(hn, waT[D_IN:, :], precision=hp)
         + ba_ref[...])
    h = jnp.maximum(z, 0.0)
    h_ref[...] = h

    weT = weT_ref[...]
    hs_ref[...] = jnp.dot(h, weT[:D_OUT, :], precision=hp)
    hd_ref[...] = jnp.dot(h, weT[D_OUT:, :], precision=hp) + be_ref[...]


ECH = 40
NECH = EPW // ECH
EROWS = 2 * ECH


def _edge_body(hs_hbm, hd_hbm, src_hbm, dst_hbm, out_hbm,
               sidx_v, didx_v, bsA, bdA, bsB, bdB, sgA, sgB, swA, swB):
    c = lax.axis_index("c")
    s = lax.axis_index("s")
    wid = c * NS + s

    pltpu.sync_copy(src_hbm.at[wid], sidx_v)
    pltpu.sync_copy(dst_hbm.at[wid], didx_v)
    out_base = 2 * wid * EPW

    def gathers(j, bs, bd, sg):
        pltpu.async_copy(hs_hbm.at[sidx_v.at[j]], bs, sg)
        pltpu.async_copy(hd_hbm.at[didx_v.at[j]], bd, sg)

    def wait_gathers(bs, bd, sg):
        pltpu.make_async_copy(hs_hbm.at[pl.ds(0, EROWS)], bs, sg).wait()
        pltpu.make_async_copy(hs_hbm.at[pl.ds(0, EROWS)], bd, sg).wait()

    def wait_write(bs, sw):
        pltpu.make_async_copy(bs, out_hbm.at[pl.ds(0, EROWS)], sw).wait()

    def add(bs, bd):
        def addrow(r, inner):
            for k in range(128 // 16):
                sl = pl.ds(k * 16, 16)
                bs[r, sl] = bs[r, sl] + bd[r, sl]
            return inner

        lax.fori_loop(0, EROWS, addrow, None)

    T2 = NECH // 2
    gathers(0, bsA, bdA, sgA)

    def body(t, carry):
        j0 = 2 * t
        j1 = j0 + 1

        @pl.when(t > 0)
        def _():
            wait_write(bsB, swB)

        gathers(j1, bsB, bdB, sgB)
        wait_gathers(bsA, bdA, sgA)
        add(bsA, bdA)
        wA = pltpu.async_copy(
            bsA, out_hbm.at[pl.ds(out_base + j0 * EROWS, EROWS)], swA)
        wait_gathers(bsB, bdB, sgB)
        add(bsB, bdB)
        wA.wait()

        @pl.when(t < T2 - 1)
        def _():
            gathers(j0 + 2, bsA, bdA, sgA)

        pltpu.async_copy(
            bsB, out_hbm.at[pl.ds(out_base + j1 * EROWS, EROWS)], swB)
        return carry

    lax.fori_loop(0, T2, body, None)
    wait_write(bsB, swB)


def kernel(nfeats, efeats, edge_index, W_apply_w, W_apply_b, W_edge_w, W_edge_b):
    nf = nfeats.reshape(N, D_IN)
    ef_pad = jnp.concatenate(
        [efeats.reshape(E, D_E),
         jnp.ones((E, 1), jnp.float32),
         jnp.zeros((E, ACC_W - D_E - 1), jnp.float32)],
        axis=1).reshape(NW, NCHUNK, CHUNK, ACC_W)
    src3 = edge_index[0].reshape(NW, NCHUNK, CHUNK)
    dst3 = edge_index[1].reshape(NW, NCHUNK, CHUNK)
    waT = W_apply_w.T
    weT = W_edge_w.T
    ba = W_apply_b.reshape(1, D_OUT)
    be = W_edge_b.reshape(1, D_EDGE)

    mesh = plsc.VectorSubcoreMesh(core_axis_name="c", subcore_axis_name="s")

    seg = pl.kernel(
        _seg_body,
        out_type=jax.ShapeDtypeStruct((NC, NS, ZCH, CHUNK, ACC_W), jnp.float32),
        mesh=mesh,
        scratch_types=[
            pltpu.VMEM((NCHUNK, CHUNK), jnp.int32),
            pltpu.VMEM((CHUNK, ACC_W), jnp.float32),
            pltpu.VMEM((ZCH, CHUNK), jnp.int32),
            pltpu.VMEM_SHARED((N_PAD, ACC_W), jnp.float32),
        ],
    )
    parts = seg(ef_pad, dst3).reshape(NC, N_PAD, ACC_W)[:, :N]

    BLK = 2000
    grid = (N // BLK,)
    h, hs, hd = pl.pallas_call(
        _tc_body,
        grid=grid,
        in_specs=[
            pl.BlockSpec((BLK, D_IN), lambda i: (i, 0)),
            pl.BlockSpec((NC, BLK, ACC_W), lambda i: (0, i, 0)),
            pl.BlockSpec((D_IN + D_E, D_OUT), lambda i: (0, 0)),
            pl.BlockSpec((1, D_OUT), lambda i: (0, 0)),
            pl.BlockSpec((D_EDGE, D_EDGE), lambda i: (0, 0)),
            pl.BlockSpec((1, D_EDGE), lambda i: (0, 0)),
        ],
        out_specs=[
            pl.BlockSpec((BLK, D_OUT), lambda i: (i, 0)),
            pl.BlockSpec((BLK, D_EDGE), lambda i: (i, 0)),
            pl.BlockSpec((BLK, D_EDGE), lambda i: (i, 0)),
        ],
        out_shape=[
            jax.ShapeDtypeStruct((N, D_OUT), jnp.float32),
            jax.ShapeDtypeStruct((N, D_EDGE), jnp.float32),
            jax.ShapeDtypeStruct((N, D_EDGE), jnp.float32),
        ],
    )(nf, parts, waT, ba, weT, be)

    hs2 = hs.reshape(2 * N, 128)
    hd2 = hd.reshape(2 * N, 128)
    two = jnp.arange(2, dtype=jnp.int32)
    srcx = (2 * edge_index[0][:, None] + two).reshape(NW, NECH, EROWS)
    dstx = (2 * edge_index[1][:, None] + two).reshape(NW, NECH, EROWS)

    edge = pl.kernel(
        _edge_body,
        out_type=jax.ShapeDtypeStruct((2 * E, 128), jnp.float32),
        mesh=mesh,
        scratch_types=[
            pltpu.VMEM((NECH, EROWS), jnp.int32),
            pltpu.VMEM((NECH, EROWS), jnp.int32),
            pltpu.VMEM((EROWS, 128), jnp.float32),
            pltpu.VMEM((EROWS, 128), jnp.float32),
            pltpu.VMEM((EROWS, 128), jnp.float32),
            pltpu.VMEM((EROWS, 128), jnp.float32),
            pltpu.SemaphoreType.DMA,
            pltpu.SemaphoreType.DMA,
            pltpu.SemaphoreType.DMA,
            pltpu.SemaphoreType.DMA,
        ],
    )(hs2, hd2, srcx, dstx)

    return h.reshape(N, 1, D_OUT), edge.reshape(E, 1, D_EDGE)
